# baseline (speedup 1.0000x reference)
import functools

import jax
import jax.numpy as jnp
from jax import lax
from jax.experimental import pallas as pl
from jax.experimental.pallas import tpu as pltpu
from jax.experimental.pallas import tpu_sc as plsc

_RADIUS = 0.2
_S = 32
_C = 128
_O = 128
_B = 4
_N = 2048
_K = 8

_TN = 256
_NT = _N // _TN
_G = _B * _NT

_NC = 2
_NS = 16
_NW = _NC * _NS
_PW = (_B * _N) // _NW
_CH = 16
_NCHUNK = _PW // _CH

_ZROW = _B * _N * _K


_RB = 256


def _tc_body(pcs_ref, pcst_ref, xt_ref, wr_ref, y_ref, gidx_ref, u_ref):
    g = pl.program_id(0)
    b = g // _NT
    t = g % _NT

    @pl.when(g == 0)
    def _():
        r = lax.broadcasted_iota(jnp.int32, (_RB, _RB), 0)
        c = lax.broadcasted_iota(jnp.int32, (_RB, _RB), 1)
        u_ref[...] = (r < c).astype(jnp.bfloat16)

    p_all = pcs_ref[0]
    p_t = pcst_ref[0]

    sq_row = p_all[0:1] * p_all[0:1] + p_all[1:2] * p_all[1:2] \
        + p_all[2:3] * p_all[2:3]
    c0 = p_t[:, 0:1]
    c1 = p_t[:, 1:2]
    c2 = p_t[:, 2:3]
    sq_t = c0 * c0 + c1 * c1 + c2 * c2
    inner = lax.dot_general(p_t, p_all, (((1,), (0,)), ((), ())))
    d2 = (sq_t + sq_row) - 2.0 * inner

    jidx = lax.broadcasted_iota(jnp.int32, (_TN, _N), 1)
    nrow = t * _TN + lax.broadcasted_iota(jnp.int32, (_TN, _N), 0)
    m = (d2 < (_RADIUS * _RADIUS)) & (jidx != nrow)

    mb = m.astype(jnp.bfloat16)
    jblk = lax.broadcasted_iota(jnp.int32, (_TN, _RB), 1)
    off = jnp.zeros((_TN, 1), jnp.float32)
    kbs = []
    for tau in range(_N // _RB):
        sl = slice(tau * _RB, (tau + 1) * _RB)
        mb_t = mb[:, sl]
        rank_t = lax.dot_general(mb_t, u_ref[...], (((1,), (0,)), ((), ())),
                                 preferred_element_type=jnp.float32)
        acc_t = m[:, sl] & (rank_t <= (float(_S - 2) - off))
        kbs.append(jnp.where(acc_t, jblk + tau * _RB, _N))
        off = off + (rank_t[:, _RB - 1:_RB]
                     + mb_t[:, _RB - 1:_RB].astype(jnp.float32))
    kb = jnp.concatenate(kbs, axis=1)

    sx = p_all[0:1] > c0
    sy = p_all[1:2] > c1
    sz = p_all[2:3] > c2

    ncol = t * _TN + lax.broadcasted_iota(jnp.int32, (_TN, 1), 0)
    cols = [(b * _N + ncol) * _K]
    for o in range(1, 8):
        ex = sx if (o & 4) else jnp.logical_not(sx)
        ey = sy if (o & 2) else jnp.logical_not(sy)
        ez = sz if (o & 1) else jnp.logical_not(sz)
        key = jnp.where(ex & ey & ez, kb, _N)
        first = jnp.min(key, axis=1, keepdims=True)
        cols.append(jnp.where(first < _N, (b * _N + first) * _K + o, _ZROW))
    gidx_ref[...] = jnp.concatenate(cols, axis=1)

    y_ref[...] = lax.dot_general(xt_ref[...], wr_ref[...],
                                 (((1,), (0,)), ((), ())),
                                 preferred_element_type=jnp.float32,
                                 precision=lax.Precision.HIGHEST)


def _tc_call(xt, pcs, pcst, wr, interpret=False):
    return pl.pallas_call(
        _tc_body,
        grid=(_G,),
        in_specs=[
            pl.BlockSpec((1, 3, _N), lambda g: (g // _NT, 0, 0)),
            pl.BlockSpec((1, _TN, 3), lambda g: (g // _NT, g % _NT, 0)),
            pl.BlockSpec((_TN, _C), lambda g: (g, 0)),
            pl.BlockSpec((_C, _K * _O), lambda g: (0, 0)),
        ],
        out_specs=[
            pl.BlockSpec((_TN, _K * _O), lambda g: (g, 0)),
            pl.BlockSpec((_TN, _K), lambda g: (g, 0)),
        ],
        out_shape=[
            jax.ShapeDtypeStruct((_B * _N, _K * _O), jnp.float32),
            jax.ShapeDtypeStruct((_B * _N, _K), jnp.int32),
        ],
        scratch_shapes=[pltpu.VMEM((_RB, _RB), jnp.bfloat16)],
        interpret=interpret,
    )(pcs, pcst, xt, wr)


def _sc_gather_accum(y2, gidx3):
    mesh = plsc.VectorSubcoreMesh(core_axis_name="c", subcore_axis_name="s")

    @functools.partial(
        pl.kernel,
        mesh=mesh,
        out_type=jax.ShapeDtypeStruct((_B * _N, _O), jnp.float32),
        scratch_types=[
            pltpu.VMEM((_NCHUNK, _CH * _K), jnp.int32),
            pltpu.VMEM((_CH * _K, _O), jnp.float32),
            pltpu.VMEM((_CH * _K, _O), jnp.float32),
            pltpu.VMEM((_CH, _O), jnp.float32),
            pltpu.VMEM((_CH, _O), jnp.float32),
            pltpu.SemaphoreType.DMA,
            pltpu.SemaphoreType.DMA,
            pltpu.SemaphoreType.DMA,
            pltpu.SemaphoreType.DMA,
        ],
    )
    def k(y2_hbm, gidx_hbm, out_hbm, idx_v, rows0, rows1, acc0, acc1,
          gsem0, gsem1, osem0, osem1):
        wid = lax.axis_index("s") * _NC + lax.axis_index("c")
        base = wid * _PW
        pltpu.sync_copy(gidx_hbm.at[wid], idx_v)

        def accum(rv, av):
            for i in range(_CH):
                for cb in range(_O // 16):
                    a = rv[i * _K, pl.ds(cb * 16, 16)]
                    for kk in range(1, _K):
                        a = a + rv[i * _K + kk, pl.ds(cb * 16, 16)]
                    av[i, pl.ds(cb * 16, 16)] = a

        def wait_gather(rv, sem):
            pltpu.make_async_copy(y2_hbm.at[pl.ds(0, _CH * _K)], rv, sem).wait()

        def wait_store(av, sem):
            pltpu.make_async_copy(av, out_hbm.at[pl.ds(0, _CH)], sem).wait()

        pltpu.async_copy(y2_hbm.at[idx_v.at[0]], rows0, gsem0)

        def pair_body(q, carry):
            ch0 = 2 * q
            pltpu.async_copy(y2_hbm.at[idx_v.at[ch0 + 1]], rows1, gsem1)
            wait_gather(rows0, gsem0)

            @pl.when(q > 0)
            def _():
                wait_store(acc0, osem0)

            accum(rows0, acc0)
            pltpu.async_copy(acc0, out_hbm.at[pl.ds(base + ch0 * _CH, _CH)],
                             osem0)

            @pl.when(q < _NCHUNK // 2 - 1)
            def _():
                pltpu.async_copy(y2_hbm.at[idx_v.at[ch0 + 2]], rows0, gsem0)

            wait_gather(rows1, gsem1)

            @pl.when(q > 0)
            def _():
                wait_store(acc1, osem1)

            accum(rows1, acc1)
            pltpu.async_copy(
                acc1, out_hbm.at[pl.ds(base + (ch0 + 1) * _CH, _CH)], osem1)
            return carry

        lax.fori_loop(0, _NCHUNK // 2, pair_body, 0)
        wait_store(acc0, osem0)
        wait_store(acc1, osem1)

    return k(y2, gidx3)


def kernel(x, pcs, W, b):
    B_, C_, N_ = x.shape
    xt = x.transpose(0, 2, 1).reshape(B_ * N_, C_)
    pcst = pcs.transpose(0, 2, 1)
    wsel = W[:, :, jnp.array([0, 2, 3, 4, 5, 6, 7, 8])]
    wr = wsel.transpose(1, 2, 0).reshape(C_, _K * _O)
    y, gidx = _tc_call(xt, pcs, pcst, wr)
    y2 = jnp.concatenate(
        [y.reshape(B_ * N_ * _K, _O), jnp.zeros((8, _O), jnp.float32)], axis=0)
    outf = _sc_gather_accum(y2, gidx.reshape(_NW, _NCHUNK, _CH * _K))
    return (outf.reshape(B_, N_, _O).transpose(0, 2, 1) + b[None, :, None])


# speedup vs baseline: 1.1118x; 1.1118x over previous
import functools

import jax
import jax.numpy as jnp
from jax import lax
from jax.experimental import pallas as pl
from jax.experimental.pallas import tpu as pltpu
from jax.experimental.pallas import tpu_sc as plsc

_RADIUS = 0.2
_S = 32
_C = 128
_O = 128
_B = 4
_N = 2048
_K = 8

_TN = 256
_NT = _N // _TN
_G = _B * _NT

_NC = 2
_NS = 16
_NW = _NC * _NS
_PW = (_B * _N) // _NW
_CH = 16
_NCHUNK = _PW // _CH

_ZROW = _B * _N * _K


_RB = 256


def _tc_body(pcs_ref, pcst_ref, xt_ref, wr_ref, y_ref, gidx_ref, u_ref):
    g = pl.program_id(0)
    b = g // _NT
    t = g % _NT

    @pl.when(g == 0)
    def _():
        r = lax.broadcasted_iota(jnp.int32, (_RB, _RB), 0)
        c = lax.broadcasted_iota(jnp.int32, (_RB, _RB), 1)
        u_ref[...] = (r < c).astype(jnp.bfloat16)

    p_all = pcs_ref[0]
    p_t = pcst_ref[0]

    sq_row = p_all[0:1] * p_all[0:1] + p_all[1:2] * p_all[1:2] \
        + p_all[2:3] * p_all[2:3]
    c0 = p_t[:, 0:1]
    c1 = p_t[:, 1:2]
    c2 = p_t[:, 2:3]
    sq_t = c0 * c0 + c1 * c1 + c2 * c2
    inner = lax.dot_general(p_t, p_all, (((1,), (0,)), ((), ())))
    d2 = (sq_t + sq_row) - 2.0 * inner

    jidx = lax.broadcasted_iota(jnp.int32, (_TN, _N), 1)
    nrow = t * _TN + lax.broadcasted_iota(jnp.int32, (_TN, 1), 0)
    m = (d2 < (_RADIUS * _RADIUS)) & (jidx != nrow)

    mb = m.astype(jnp.bfloat16)
    jblk = lax.broadcasted_iota(jnp.int32, (_TN, _RB), 1)
    off = jnp.zeros((_TN, 1), jnp.float32)
    kbs = []
    for tau in range(_N // _RB):
        sl = slice(tau * _RB, (tau + 1) * _RB)
        mb_t = mb[:, sl]
        rank_t = lax.dot_general(mb_t, u_ref[...], (((1,), (0,)), ((), ())),
                                 preferred_element_type=jnp.float32)
        acc_t = m[:, sl] & (rank_t <= (float(_S - 2) - off))
        kbs.append(jnp.where(acc_t, jblk + tau * _RB, _N))
        off = off + (rank_t[:, _RB - 1:_RB]
                     + mb_t[:, _RB - 1:_RB].astype(jnp.float32))
    kb = jnp.concatenate(kbs, axis=1)

    octv = ((p_all[0:1] > c0).astype(jnp.int32) * (4 << 12)
            + (p_all[1:2] > c1).astype(jnp.int32) * (2 << 12)
            + (p_all[2:3] > c2).astype(jnp.int32) * (1 << 12))
    kb3 = kb + octv

    ncol = t * _TN + lax.broadcasted_iota(jnp.int32, (_TN, 1), 0)
    cols = [(b * _N + ncol) * _K]
    for o in range(1, 8):
        key = lax.bitwise_xor(kb3, jnp.int32(o << 12))
        part = key[:, 0:128]
        for s in range(1, _N // 128):
            part = jnp.minimum(part, key[:, s * 128:(s + 1) * 128])
        first = jnp.min(part, axis=1, keepdims=True)
        cols.append(jnp.where(first < _N, (b * _N + first) * _K + o, _ZROW))
    gidx_ref[...] = jnp.concatenate(cols, axis=1)

    y_ref[...] = lax.dot_general(xt_ref[...], wr_ref[...],
                                 (((1,), (0,)), ((), ())),
                                 preferred_element_type=jnp.float32,
                                 precision=lax.Precision.HIGHEST)


def _tc_call(xt, pcs, pcst, wr, interpret=False):
    return pl.pallas_call(
        _tc_body,
        grid=(_G,),
        in_specs=[
            pl.BlockSpec((1, 3, _N), lambda g: (g // _NT, 0, 0)),
            pl.BlockSpec((1, _TN, 3), lambda g: (g // _NT, g % _NT, 0)),
            pl.BlockSpec((_TN, _C), lambda g: (g, 0)),
            pl.BlockSpec((_C, _K * _O), lambda g: (0, 0)),
        ],
        out_specs=[
            pl.BlockSpec((_TN, _K * _O), lambda g: (g, 0)),
            pl.BlockSpec((_TN, _K), lambda g: (g, 0)),
        ],
        out_shape=[
            jax.ShapeDtypeStruct((_B * _N, _K * _O), jnp.float32),
            jax.ShapeDtypeStruct((_B * _N, _K), jnp.int32),
        ],
        scratch_shapes=[pltpu.VMEM((_RB, _RB), jnp.bfloat16)],
        interpret=interpret,
    )(pcs, pcst, xt, wr)


def _sc_gather_accum(y2, gidx3):
    mesh = plsc.VectorSubcoreMesh(core_axis_name="c", subcore_axis_name="s")

    @functools.partial(
        pl.kernel,
        mesh=mesh,
        out_type=jax.ShapeDtypeStruct((_B * _N, _O), jnp.float32),
        scratch_types=[
            pltpu.VMEM((_NCHUNK, _CH * _K), jnp.int32),
            pltpu.VMEM((_CH * _K, _O), jnp.float32),
            pltpu.VMEM((_CH * _K, _O), jnp.float32),
            pltpu.VMEM((_CH, _O), jnp.float32),
            pltpu.VMEM((_CH, _O), jnp.float32),
            pltpu.SemaphoreType.DMA,
            pltpu.SemaphoreType.DMA,
            pltpu.SemaphoreType.DMA,
            pltpu.SemaphoreType.DMA,
        ],
    )
    def k(y2_hbm, gidx_hbm, out_hbm, idx_v, rows0, rows1, acc0, acc1,
          gsem0, gsem1, osem0, osem1):
        wid = lax.axis_index("s") * _NC + lax.axis_index("c")
        base = wid * _PW
        pltpu.sync_copy(gidx_hbm.at[wid], idx_v)

        def accum(rv, av):
            for i in range(_CH):
                for cb in range(_O // 16):
                    a = rv[i * _K, pl.ds(cb * 16, 16)]
                    for kk in range(1, _K):
                        a = a + rv[i * _K + kk, pl.ds(cb * 16, 16)]
                    av[i, pl.ds(cb * 16, 16)] = a

        def wait_gather(rv, sem):
            pltpu.make_async_copy(y2_hbm.at[pl.ds(0, _CH * _K)], rv, sem).wait()

        def wait_store(av, sem):
            pltpu.make_async_copy(av, out_hbm.at[pl.ds(0, _CH)], sem).wait()

        pltpu.async_copy(y2_hbm.at[idx_v.at[0]], rows0, gsem0)

        def pair_body(q, carry):
            ch0 = 2 * q
            pltpu.async_copy(y2_hbm.at[idx_v.at[ch0 + 1]], rows1, gsem1)
            wait_gather(rows0, gsem0)

            @pl.when(q > 0)
            def _():
                wait_store(acc0, osem0)

            accum(rows0, acc0)
            pltpu.async_copy(acc0, out_hbm.at[pl.ds(base + ch0 * _CH, _CH)],
                             osem0)

            @pl.when(q < _NCHUNK // 2 - 1)
            def _():
                pltpu.async_copy(y2_hbm.at[idx_v.at[ch0 + 2]], rows0, gsem0)

            wait_gather(rows1, gsem1)

            @pl.when(q > 0)
            def _():
                wait_store(acc1, osem1)

            accum(rows1, acc1)
            pltpu.async_copy(
                acc1, out_hbm.at[pl.ds(base + (ch0 + 1) * _CH, _CH)], osem1)
            return carry

        lax.fori_loop(0, _NCHUNK // 2, pair_body, 0)
        wait_store(acc0, osem0)
        wait_store(acc1, osem1)

    return k(y2, gidx3)


def kernel(x, pcs, W, b):
    B_, C_, N_ = x.shape
    xt = x.transpose(0, 2, 1).reshape(B_ * N_, C_)
    pcst = pcs.transpose(0, 2, 1)
    wsel = W[:, :, jnp.array([0, 2, 3, 4, 5, 6, 7, 8])]
    wr = wsel.transpose(1, 2, 0).reshape(C_, _K * _O)
    y, gidx = _tc_call(xt, pcs, pcst, wr)
    y2 = jnp.concatenate(
        [y.reshape(B_ * N_ * _K, _O), jnp.zeros((8, _O), jnp.float32)], axis=0)
    outf = _sc_gather_accum(y2, gidx.reshape(_NW, _NCHUNK, _CH * _K))
    return (outf.reshape(B_, N_, _O).transpose(0, 2, 1) + b[None, :, None])


# speedup vs baseline: 1.2951x; 1.1649x over previous
import functools

import jax
import jax.numpy as jnp
from jax import lax
from jax.experimental import pallas as pl
from jax.experimental.pallas import tpu as pltpu
from jax.experimental.pallas import tpu_sc as plsc

_RADIUS = 0.2
_S = 32
_C = 128
_O = 128
_B = 4
_N = 2048
_K = 8

_TN = 256
_NT = _N // _TN
_G = _B * _NT

_NC = 2
_NS = 16
_NW = _NC * _NS
_PW = None
_CH = 16

_ZROW = None


_RB = 256
_HB = 2
_PW = (_HB * _N) // _NW
_NCHUNK = _PW // _CH
_ZROW = _HB * _N * _K


def _tc_body(pcs_ref, pcst_ref, xt_ref, wr_ref, y_ref, gidx_ref, u_ref):
    g = pl.program_id(0)
    b = g // _NT
    t = g % _NT

    @pl.when(g == 0)
    def _():
        r = lax.broadcasted_iota(jnp.int32, (_RB, _RB), 0)
        c = lax.broadcasted_iota(jnp.int32, (_RB, _RB), 1)
        u_ref[...] = (r < c).astype(jnp.bfloat16)

    p_all = pcs_ref[0]
    p_t = pcst_ref[0]

    sq_row = p_all[0:1] * p_all[0:1] + p_all[1:2] * p_all[1:2] \
        + p_all[2:3] * p_all[2:3]
    c0 = p_t[:, 0:1]
    c1 = p_t[:, 1:2]
    c2 = p_t[:, 2:3]
    sq_t = c0 * c0 + c1 * c1 + c2 * c2
    inner = lax.dot_general(p_t, p_all, (((1,), (0,)), ((), ())))
    d2 = (sq_t + sq_row) - 2.0 * inner

    jidx = lax.broadcasted_iota(jnp.int32, (_TN, _N), 1)
    nrow = t * _TN + lax.broadcasted_iota(jnp.int32, (_TN, 1), 0)
    m = (d2 < (_RADIUS * _RADIUS)) & (jidx != nrow)

    mb = m.astype(jnp.bfloat16)
    jblk = lax.broadcasted_iota(jnp.int32, (_TN, _RB), 1)
    off = jnp.zeros((_TN, 1), jnp.float32)
    kbs = []
    for tau in range(_N // _RB):
        sl = slice(tau * _RB, (tau + 1) * _RB)
        mb_t = mb[:, sl]
        rank_t = lax.dot_general(mb_t, u_ref[...], (((1,), (0,)), ((), ())),
                                 preferred_element_type=jnp.float32)
        acc_t = m[:, sl] & (rank_t <= (float(_S - 2) - off))
        kbs.append(jnp.where(acc_t, jblk + tau * _RB, _N))
        off = off + (rank_t[:, _RB - 1:_RB]
                     + mb_t[:, _RB - 1:_RB].astype(jnp.float32))
    kb = jnp.concatenate(kbs, axis=1)

    octv = ((p_all[0:1] > c0).astype(jnp.int32) * (4 << 12)
            + (p_all[1:2] > c1).astype(jnp.int32) * (2 << 12)
            + (p_all[2:3] > c2).astype(jnp.int32) * (1 << 12))
    kb3 = kb + octv

    ncol = t * _TN + lax.broadcasted_iota(jnp.int32, (_TN, 1), 0)
    cols = [(b * _N + ncol) * _K]
    for o in range(1, 8):
        key = lax.bitwise_xor(kb3, jnp.int32(o << 12))
        part = key[:, 0:128]
        for s in range(1, _N // 128):
            part = jnp.minimum(part, key[:, s * 128:(s + 1) * 128])
        first = jnp.min(part, axis=1, keepdims=True)
        cols.append(jnp.where(first < _N, (b * _N + first) * _K + o, _ZROW))
    gidx_ref[...] = jnp.concatenate(cols, axis=1)

    y_ref[...] = lax.dot_general(xt_ref[...], wr_ref[...],
                                 (((1,), (0,)), ((), ())),
                                 preferred_element_type=jnp.float32,
                                 precision=lax.Precision.HIGHEST)


def _tc_call(xt, pcs, pcst, wr, interpret=False):
    return pl.pallas_call(
        _tc_body,
        grid=(_HB * _NT,),
        in_specs=[
            pl.BlockSpec((1, 3, _N), lambda g: (g // _NT, 0, 0)),
            pl.BlockSpec((1, _TN, 3), lambda g: (g // _NT, g % _NT, 0)),
            pl.BlockSpec((_TN, _C), lambda g: (g, 0)),
            pl.BlockSpec((_C, _K * _O), lambda g: (0, 0)),
        ],
        out_specs=[
            pl.BlockSpec((_TN, _K * _O), lambda g: (g, 0)),
            pl.BlockSpec((_TN, _K), lambda g: (g, 0)),
        ],
        out_shape=[
            jax.ShapeDtypeStruct((_HB * _N, _K * _O), jnp.float32),
            jax.ShapeDtypeStruct((_HB * _N, _K), jnp.int32),
        ],
        scratch_shapes=[pltpu.VMEM((_RB, _RB), jnp.bfloat16)],
        interpret=interpret,
    )(pcs, pcst, xt, wr)


def _sc_gather_accum(y2, gidx3):
    mesh = plsc.VectorSubcoreMesh(core_axis_name="c", subcore_axis_name="s")

    @functools.partial(
        pl.kernel,
        mesh=mesh,
        out_type=jax.ShapeDtypeStruct((_HB * _N, _O), jnp.float32),
        scratch_types=[
            pltpu.VMEM((_NCHUNK, _CH * _K), jnp.int32),
            pltpu.VMEM((_CH * _K, _O), jnp.float32),
            pltpu.VMEM((_CH * _K, _O), jnp.float32),
            pltpu.VMEM((_CH, _O), jnp.float32),
            pltpu.VMEM((_CH, _O), jnp.float32),
            pltpu.SemaphoreType.DMA,
            pltpu.SemaphoreType.DMA,
            pltpu.SemaphoreType.DMA,
            pltpu.SemaphoreType.DMA,
        ],
    )
    def k(y2_hbm, gidx_hbm, out_hbm, idx_v, rows0, rows1, acc0, acc1,
          gsem0, gsem1, osem0, osem1):
        wid = lax.axis_index("s") * _NC + lax.axis_index("c")
        base = wid * _PW
        pltpu.sync_copy(gidx_hbm.at[wid], idx_v)

        def accum(rv, av):
            for i in range(_CH):
                for cb in range(_O // 16):
                    a = rv[i * _K, pl.ds(cb * 16, 16)]
                    for kk in range(1, _K):
                        a = a + rv[i * _K + kk, pl.ds(cb * 16, 16)]
                    av[i, pl.ds(cb * 16, 16)] = a

        def wait_gather(rv, sem):
            pltpu.make_async_copy(y2_hbm.at[pl.ds(0, _CH * _K)], rv, sem).wait()

        def wait_store(av, sem):
            pltpu.make_async_copy(av, out_hbm.at[pl.ds(0, _CH)], sem).wait()

        pltpu.async_copy(y2_hbm.at[idx_v.at[0]], rows0, gsem0)

        def pair_body(q, carry):
            ch0 = 2 * q
            pltpu.async_copy(y2_hbm.at[idx_v.at[ch0 + 1]], rows1, gsem1)
            wait_gather(rows0, gsem0)

            @pl.when(q > 0)
            def _():
                wait_store(acc0, osem0)

            accum(rows0, acc0)
            pltpu.async_copy(acc0, out_hbm.at[pl.ds(base + ch0 * _CH, _CH)],
                             osem0)

            @pl.when(q < _NCHUNK // 2 - 1)
            def _():
                pltpu.async_copy(y2_hbm.at[idx_v.at[ch0 + 2]], rows0, gsem0)

            wait_gather(rows1, gsem1)

            @pl.when(q > 0)
            def _():
                wait_store(acc1, osem1)

            accum(rows1, acc1)
            pltpu.async_copy(
                acc1, out_hbm.at[pl.ds(base + (ch0 + 1) * _CH, _CH)], osem1)
            return carry

        lax.fori_loop(0, _NCHUNK // 2, pair_body, 0)
        wait_store(acc0, osem0)
        wait_store(acc1, osem1)

    return k(y2, gidx3)


def kernel(x, pcs, W, b):
    B_, C_, N_ = x.shape
    xt = x.transpose(0, 2, 1).reshape(B_ * N_, C_)
    pcst = pcs.transpose(0, 2, 1)
    wsel = W[:, :, jnp.array([0, 2, 3, 4, 5, 6, 7, 8])]
    wr = wsel.transpose(1, 2, 0).reshape(C_, _K * _O)
    outs = []
    for h in range(B_ // _HB):
        bs = slice(h * _HB, (h + 1) * _HB)
        xth = xt.reshape(B_, N_, C_)[bs].reshape(_HB * N_, C_)
        y, gidx = _tc_call(xth, pcs[bs], pcst[bs], wr)
        y2 = jnp.concatenate(
            [y.reshape(_HB * N_ * _K, _O), jnp.zeros((8, _O), jnp.float32)],
            axis=0)
        outs.append(
            _sc_gather_accum(y2, gidx.reshape(_NW, _NCHUNK, _CH * _K)))
    outf = jnp.concatenate(outs, axis=0)
    return (outf.reshape(B_, N_, _O).transpose(0, 2, 1) + b[None, :, None])


# speedup vs baseline: 1.3051x; 1.0077x over previous
import functools

import jax
import jax.numpy as jnp
from jax import lax
from jax.experimental import pallas as pl
from jax.experimental.pallas import tpu as pltpu
from jax.experimental.pallas import tpu_sc as plsc

_RADIUS = 0.2
_S = 32
_C = 128
_O = 128
_B = 4
_N = 2048
_K = 8

_TN = 256
_NT = _N // _TN
_G = _B * _NT

_NC = 2
_NS = 16
_NW = _NC * _NS
_PW = None
_CH = 16

_ZROW = None


_RB = 256
_HB = 2
_PW = (_HB * _N) // _NW
_NCHUNK = _PW // _CH
_ZROW = _HB * _N * _K


def _tc_body(pcs_ref, pcst_ref, xt_ref, wr_ref, y_ref, gidx_ref, u_ref):
    g = pl.program_id(0)
    b = g // _NT
    t = g % _NT

    @pl.when(g == 0)
    def _():
        r = lax.broadcasted_iota(jnp.int32, (_RB, _RB), 0)
        c = lax.broadcasted_iota(jnp.int32, (_RB, _RB), 1)
        u_ref[...] = (r < c).astype(jnp.bfloat16)

    p_all = pcs_ref[0]
    p_t = pcst_ref[0]

    sq_row = p_all[0:1] * p_all[0:1] + p_all[1:2] * p_all[1:2] \
        + p_all[2:3] * p_all[2:3]
    c0 = p_t[:, 0:1]
    c1 = p_t[:, 1:2]
    c2 = p_t[:, 2:3]
    sq_t = c0 * c0 + c1 * c1 + c2 * c2
    inner = lax.dot_general(p_t, p_all, (((1,), (0,)), ((), ())))
    d2 = (sq_t + sq_row) - 2.0 * inner

    jidx = lax.broadcasted_iota(jnp.int32, (_TN, _N), 1)
    nrow = t * _TN + lax.broadcasted_iota(jnp.int32, (_TN, 1), 0)
    m = (d2 < (_RADIUS * _RADIUS)) & (jidx != nrow)

    mb = m.astype(jnp.bfloat16)
    jblk = lax.broadcasted_iota(jnp.int32, (_TN, _RB), 1)
    off = jnp.zeros((_TN, 1), jnp.float32)
    kbs = []
    for tau in range(_N // _RB):
        sl = slice(tau * _RB, (tau + 1) * _RB)
        mb_t = mb[:, sl]
        rank_t = lax.dot_general(mb_t, u_ref[...], (((1,), (0,)), ((), ())),
                                 preferred_element_type=jnp.float32)
        acc_t = m[:, sl] & (rank_t <= (float(_S - 2) - off))
        kbs.append(jnp.where(acc_t, jblk + tau * _RB, _N))
        off = off + (rank_t[:, _RB - 1:_RB]
                     + mb_t[:, _RB - 1:_RB].astype(jnp.float32))
    kb = jnp.concatenate(kbs, axis=1)

    octv = ((p_all[0:1] > c0).astype(jnp.int32) * (4 << 12)
            + (p_all[1:2] > c1).astype(jnp.int32) * (2 << 12)
            + (p_all[2:3] > c2).astype(jnp.int32) * (1 << 12))
    kb3 = kb + octv

    ncol = t * _TN + lax.broadcasted_iota(jnp.int32, (_TN, 1), 0)
    cols = [(b * _N + ncol) * _K]
    for o in range(1, 8):
        oc = jnp.int32(o << 12)
        parts = [lax.bitwise_xor(kb3[:, s * 128:(s + 1) * 128], oc)
                 for s in range(_N // 128)]
        while len(parts) > 1:
            parts = [jnp.minimum(parts[i], parts[i + 1])
                     for i in range(0, len(parts), 2)]
        first = jnp.min(parts[0], axis=1, keepdims=True)
        cols.append(jnp.where(first < _N, (b * _N + first) * _K + o, _ZROW))
    gidx_ref[...] = jnp.concatenate(cols, axis=1)

    y_ref[...] = lax.dot_general(xt_ref[...], wr_ref[...],
                                 (((1,), (0,)), ((), ())),
                                 preferred_element_type=jnp.float32,
                                 precision=lax.Precision.HIGHEST)


def _tc_call(xt, pcs, pcst, wr, interpret=False):
    return pl.pallas_call(
        _tc_body,
        grid=(_HB * _NT,),
        in_specs=[
            pl.BlockSpec((1, 3, _N), lambda g: (g // _NT, 0, 0)),
            pl.BlockSpec((1, _TN, 3), lambda g: (g // _NT, g % _NT, 0)),
            pl.BlockSpec((_TN, _C), lambda g: (g, 0)),
            pl.BlockSpec((_C, _K * _O), lambda g: (0, 0)),
        ],
        out_specs=[
            pl.BlockSpec((_TN, _K * _O), lambda g: (g, 0)),
            pl.BlockSpec((_TN, _K), lambda g: (g, 0)),
        ],
        out_shape=[
            jax.ShapeDtypeStruct((_HB * _N, _K * _O), jnp.float32),
            jax.ShapeDtypeStruct((_HB * _N, _K), jnp.int32),
        ],
        scratch_shapes=[pltpu.VMEM((_RB, _RB), jnp.bfloat16)],
        interpret=interpret,
    )(pcs, pcst, xt, wr)


def _sc_gather_accum(y2, gidx3):
    mesh = plsc.VectorSubcoreMesh(core_axis_name="c", subcore_axis_name="s")

    @functools.partial(
        pl.kernel,
        mesh=mesh,
        out_type=jax.ShapeDtypeStruct((_HB * _N, _O), jnp.float32),
        scratch_types=[
            pltpu.VMEM((_NCHUNK, _CH * _K), jnp.int32),
            pltpu.VMEM((_CH * _K, _O), jnp.float32),
            pltpu.VMEM((_CH * _K, _O), jnp.float32),
            pltpu.VMEM((_CH, _O), jnp.float32),
            pltpu.VMEM((_CH, _O), jnp.float32),
            pltpu.SemaphoreType.DMA,
            pltpu.SemaphoreType.DMA,
            pltpu.SemaphoreType.DMA,
            pltpu.SemaphoreType.DMA,
        ],
    )
    def k(y2_hbm, gidx_hbm, out_hbm, idx_v, rows0, rows1, acc0, acc1,
          gsem0, gsem1, osem0, osem1):
        wid = lax.axis_index("s") * _NC + lax.axis_index("c")
        base = wid * _PW
        pltpu.sync_copy(gidx_hbm.at[wid], idx_v)

        def accum(rv, av):
            for i in range(_CH):
                for cb in range(_O // 16):
                    a = rv[i * _K, pl.ds(cb * 16, 16)]
                    for kk in range(1, _K):
                        a = a + rv[i * _K + kk, pl.ds(cb * 16, 16)]
                    av[i, pl.ds(cb * 16, 16)] = a

        def wait_gather(rv, sem):
            pltpu.make_async_copy(y2_hbm.at[pl.ds(0, _CH * _K)], rv, sem).wait()

        def wait_store(av, sem):
            pltpu.make_async_copy(av, out_hbm.at[pl.ds(0, _CH)], sem).wait()

        pltpu.async_copy(y2_hbm.at[idx_v.at[0]], rows0, gsem0)

        def pair_body(q, carry):
            ch0 = 2 * q
            pltpu.async_copy(y2_hbm.at[idx_v.at[ch0 + 1]], rows1, gsem1)
            wait_gather(rows0, gsem0)

            @pl.when(q > 0)
            def _():
                wait_store(acc0, osem0)

            accum(rows0, acc0)
            pltpu.async_copy(acc0, out_hbm.at[pl.ds(base + ch0 * _CH, _CH)],
                             osem0)

            @pl.when(q < _NCHUNK // 2 - 1)
            def _():
                pltpu.async_copy(y2_hbm.at[idx_v.at[ch0 + 2]], rows0, gsem0)

            wait_gather(rows1, gsem1)

            @pl.when(q > 0)
            def _():
                wait_store(acc1, osem1)

            accum(rows1, acc1)
            pltpu.async_copy(
                acc1, out_hbm.at[pl.ds(base + (ch0 + 1) * _CH, _CH)], osem1)
            return carry

        lax.fori_loop(0, _NCHUNK // 2, pair_body, 0)
        wait_store(acc0, osem0)
        wait_store(acc1, osem1)

    return k(y2, gidx3)


def kernel(x, pcs, W, b):
    B_, C_, N_ = x.shape
    xt = x.transpose(0, 2, 1).reshape(B_ * N_, C_)
    pcst = pcs.transpose(0, 2, 1)
    wsel = W[:, :, jnp.array([0, 2, 3, 4, 5, 6, 7, 8])]
    wr = wsel.transpose(1, 2, 0).reshape(C_, _K * _O)
    outs = []
    for h in range(B_ // _HB):
        bs = slice(h * _HB, (h + 1) * _HB)
        xth = xt.reshape(B_, N_, C_)[bs].reshape(_HB * N_, C_)
        y, gidx = _tc_call(xth, pcs[bs], pcst[bs], wr)
        y2 = jnp.concatenate(
            [y.reshape(_HB * N_ * _K, _O), jnp.zeros((8, _O), jnp.float32)],
            axis=0)
        outs.append(
            _sc_gather_accum(y2, gidx.reshape(_NW, _NCHUNK, _CH * _K)))
    outf = jnp.concatenate(outs, axis=0)
    return (outf.reshape(B_, N_, _O).transpose(0, 2, 1) + b[None, :, None])


# speedup vs baseline: 1.3349x; 1.0228x over previous
import functools

import jax
import jax.numpy as jnp
from jax import lax
from jax.experimental import pallas as pl
from jax.experimental.pallas import tpu as pltpu
from jax.experimental.pallas import tpu_sc as plsc

_RADIUS = 0.2
_S = 32
_C = 128
_O = 128
_B = 4
_N = 2048
_K = 8

_TN = 256
_NT = _N // _TN
_G = _B * _NT

_NC = 2
_NS = 16
_NW = _NC * _NS
_PW = None
_CH = 16

_ZROW = None


_RB = 256
_HB = 2
_PW = (_HB * _N) // _NW
_NCHUNK = _PW // _CH
_ZROW = _HB * _N * _K


def _tc_body(pcs_ref, pcst_ref, xt_ref, wr_ref, y_ref, gidx_ref, u_ref):
    g = pl.program_id(0)
    ge = jnp.minimum(g, _HB * _NT - 1)
    b = ge // _NT
    t = ge % _NT

    @pl.when(g == 0)
    def _():
        r = lax.broadcasted_iota(jnp.int32, (_RB, _RB), 0)
        c = lax.broadcasted_iota(jnp.int32, (_RB, _RB), 1)
        u_ref[...] = (r < c).astype(jnp.bfloat16)

    p_all = pcs_ref[0]
    p_t = pcst_ref[0]

    sq_row = p_all[0:1] * p_all[0:1] + p_all[1:2] * p_all[1:2] \
        + p_all[2:3] * p_all[2:3]
    c0 = p_t[:, 0:1]
    c1 = p_t[:, 1:2]
    c2 = p_t[:, 2:3]
    sq_t = c0 * c0 + c1 * c1 + c2 * c2
    inner = lax.dot_general(p_t, p_all, (((1,), (0,)), ((), ())))
    d2 = (sq_t + sq_row) - 2.0 * inner

    jidx = lax.broadcasted_iota(jnp.int32, (_TN, _N), 1)
    nrow = t * _TN + lax.broadcasted_iota(jnp.int32, (_TN, 1), 0)
    m = (d2 < (_RADIUS * _RADIUS)) & (jidx != nrow)

    mb = m.astype(jnp.bfloat16)
    jblk = lax.broadcasted_iota(jnp.int32, (_TN, _RB), 1)
    off = jnp.zeros((_TN, 1), jnp.float32)
    kbs = []
    for tau in range(_N // _RB):
        sl = slice(tau * _RB, (tau + 1) * _RB)
        mb_t = mb[:, sl]
        rank_t = lax.dot_general(mb_t, u_ref[...], (((1,), (0,)), ((), ())),
                                 preferred_element_type=jnp.float32)
        acc_t = m[:, sl] & (rank_t <= (float(_S - 2) - off))
        kbs.append(jnp.where(acc_t, jblk + tau * _RB, _N))
        off = off + (rank_t[:, _RB - 1:_RB]
                     + mb_t[:, _RB - 1:_RB].astype(jnp.float32))
    kb = jnp.concatenate(kbs, axis=1)

    octv = ((p_all[0:1] > c0).astype(jnp.int32) * (4 << 12)
            + (p_all[1:2] > c1).astype(jnp.int32) * (2 << 12)
            + (p_all[2:3] > c2).astype(jnp.int32) * (1 << 12))
    kb3 = kb + octv

    ncol = t * _TN + lax.broadcasted_iota(jnp.int32, (_TN, 1), 0)
    cols = [(b * _N + ncol) * _K]
    for o in range(1, 8):
        oc = jnp.int32(o << 12)
        parts = [lax.bitwise_xor(kb3[:, s * 128:(s + 1) * 128], oc)
                 for s in range(_N // 128)]
        while len(parts) > 1:
            parts = [jnp.minimum(parts[i], parts[i + 1])
                     for i in range(0, len(parts), 2)]
        first = jnp.min(parts[0], axis=1, keepdims=True)
        cols.append(jnp.where(first < _N, (b * _N + first) * _K + o, _ZROW))
    gidx_ref[...] = jnp.concatenate(cols, axis=1)

    yv = lax.dot_general(xt_ref[...], wr_ref[...],
                         (((1,), (0,)), ((), ())),
                         preferred_element_type=jnp.float32,
                         precision=lax.Precision.HIGHEST)
    y_ref[...] = jnp.where(g == _HB * _NT, jnp.zeros_like(yv), yv)


def _tc_call(xt, pcs, pcst, wr, interpret=False):
    return pl.pallas_call(
        _tc_body,
        grid=(_HB * _NT + 1,),
        in_specs=[
            pl.BlockSpec(
                (1, 3, _N),
                lambda g: (jnp.minimum(g, _HB * _NT - 1) // _NT, 0, 0)),
            pl.BlockSpec(
                (1, _TN, 3),
                lambda g: (jnp.minimum(g, _HB * _NT - 1) // _NT,
                           jnp.minimum(g, _HB * _NT - 1) % _NT, 0)),
            pl.BlockSpec((_TN, _C),
                         lambda g: (jnp.minimum(g, _HB * _NT - 1), 0)),
            pl.BlockSpec((_C, _K * _O), lambda g: (0, 0)),
        ],
        out_specs=[
            pl.BlockSpec((_TN, _K * _O), lambda g: (g, 0)),
            pl.BlockSpec((_TN, _K),
                         lambda g: (jnp.minimum(g, _HB * _NT - 1), 0)),
        ],
        out_shape=[
            jax.ShapeDtypeStruct((_HB * _N + _TN, _K * _O), jnp.float32),
            jax.ShapeDtypeStruct((_HB * _N, _K), jnp.int32),
        ],
        scratch_shapes=[pltpu.VMEM((_RB, _RB), jnp.bfloat16)],
        interpret=interpret,
    )(pcs, pcst, xt, wr)


def _sc_gather_accum(y2, gidx3):
    mesh = plsc.VectorSubcoreMesh(core_axis_name="c", subcore_axis_name="s")

    @functools.partial(
        pl.kernel,
        mesh=mesh,
        out_type=jax.ShapeDtypeStruct((_HB * _N, _O), jnp.float32),
        scratch_types=[
            pltpu.VMEM((_NCHUNK, _CH * _K), jnp.int32),
            pltpu.VMEM((_CH * _K, _O), jnp.float32),
            pltpu.VMEM((_CH * _K, _O), jnp.float32),
            pltpu.VMEM((_CH, _O), jnp.float32),
            pltpu.VMEM((_CH, _O), jnp.float32),
            pltpu.SemaphoreType.DMA,
            pltpu.SemaphoreType.DMA,
            pltpu.SemaphoreType.DMA,
            pltpu.SemaphoreType.DMA,
        ],
    )
    def k(y2_hbm, gidx_hbm, out_hbm, idx_v, rows0, rows1, acc0, acc1,
          gsem0, gsem1, osem0, osem1):
        wid = lax.axis_index("s") * _NC + lax.axis_index("c")
        base = wid * _PW
        pltpu.sync_copy(gidx_hbm.at[wid], idx_v)

        def accum(rv, av):
            for i in range(_CH):
                for cb in range(_O // 16):
                    a = rv[i * _K, pl.ds(cb * 16, 16)]
                    for kk in range(1, _K):
                        a = a + rv[i * _K + kk, pl.ds(cb * 16, 16)]
                    av[i, pl.ds(cb * 16, 16)] = a

        def wait_gather(rv, sem):
            pltpu.make_async_copy(y2_hbm.at[pl.ds(0, _CH * _K)], rv, sem).wait()

        def wait_store(av, sem):
            pltpu.make_async_copy(av, out_hbm.at[pl.ds(0, _CH)], sem).wait()

        pltpu.async_copy(y2_hbm.at[idx_v.at[0]], rows0, gsem0)

        def pair_body(q, carry):
            ch0 = 2 * q
            pltpu.async_copy(y2_hbm.at[idx_v.at[ch0 + 1]], rows1, gsem1)
            wait_gather(rows0, gsem0)

            @pl.when(q > 0)
            def _():
                wait_store(acc0, osem0)

            accum(rows0, acc0)
            pltpu.async_copy(acc0, out_hbm.at[pl.ds(base + ch0 * _CH, _CH)],
                             osem0)

            @pl.when(q < _NCHUNK // 2 - 1)
            def _():
                pltpu.async_copy(y2_hbm.at[idx_v.at[ch0 + 2]], rows0, gsem0)

            wait_gather(rows1, gsem1)

            @pl.when(q > 0)
            def _():
                wait_store(acc1, osem1)

            accum(rows1, acc1)
            pltpu.async_copy(
                acc1, out_hbm.at[pl.ds(base + (ch0 + 1) * _CH, _CH)], osem1)
            return carry

        lax.fori_loop(0, _NCHUNK // 2, pair_body, 0)
        wait_store(acc0, osem0)
        wait_store(acc1, osem1)

    return k(y2, gidx3)


def kernel(x, pcs, W, b):
    B_, C_, N_ = x.shape
    xt = x.transpose(0, 2, 1).reshape(B_ * N_, C_)
    pcst = pcs.transpose(0, 2, 1)
    wsel = W[:, :, jnp.array([0, 2, 3, 4, 5, 6, 7, 8])]
    wr = wsel.transpose(1, 2, 0).reshape(C_, _K * _O)
    outs = []
    for h in range(B_ // _HB):
        bs = slice(h * _HB, (h + 1) * _HB)
        xth = xt.reshape(B_, N_, C_)[bs].reshape(_HB * N_, C_)
        y, gidx = _tc_call(xth, pcs[bs], pcst[bs], wr)
        y2 = y.reshape((_HB * N_ + _TN) * _K, _O)
        outs.append(
            _sc_gather_accum(y2, gidx.reshape(_NW, _NCHUNK, _CH * _K)))
    outf = jnp.concatenate(outs, axis=0)
    return (outf.reshape(B_, N_, _O).transpose(0, 2, 1) + b[None, :, None])
